# baseline (speedup 1.0000x reference)
import functools

import jax
import jax.numpy as jnp
from jax import lax
from jax.experimental import pallas as pl
from jax.experimental.pallas import tpu as pltpu
from jax.experimental.pallas import tpu_sc as plsc

F32 = jnp.float32
H = 128
NC = 2
NS = 16
NW = NC * NS
FPT = H // NW


def _lk(x):
    return jnp.maximum(x, 0.01 * x)


def _elu(x):
    return jnp.where(x > 0, x, jnp.exp(x) - 1.0)


def _sig(x):
    return 1.0 / (1.0 + jnp.exp(-x))


def _dot(a, b):
    return jnp.dot(a, b, preferred_element_type=F32)


def _dotT(a, b):
    return lax.dot_general(a, b, (((0,), (0,)), ((), ())),
                           preferred_element_type=F32)


def _gru(inp, hid, wih, whh, bih, bhh):
    gi = _dot(inp, wih) + bih
    gh = _dot(hid, whh) + bhh
    r = _sig(gi[:, :H] + gh[:, :H])
    z = _sig(gi[:, H:2 * H] + gh[:, H:2 * H])
    n = jnp.tanh(gi[:, 2 * H:] + r * gh[:, 2 * H:])
    return (1.0 - z) * n + z * hid


def _tc_node_pre(x, lin1_wt, lin1_b, w1x_t, lin2_t, att_r):
    N = x.shape[0]

    def body(x_ref, w1_ref, b1_ref, wx_ref, w2_ref, ar_ref,
             x1_ref, u_ref, m_ref, r_ref):
        x1 = _lk(_dot(x_ref[...], w1_ref[...]) + b1_ref[...])
        x1_ref[...] = x1
        u_ref[...] = _dot(x1, wx_ref[...])
        m_ref[...] = _dot(x1, w2_ref[...])
        r_ref[...] = _dot(x1, ar_ref[...])

    return pl.pallas_call(
        body,
        out_shape=[
            jax.ShapeDtypeStruct((N, H), F32),
            jax.ShapeDtypeStruct((N, H), F32),
            jax.ShapeDtypeStruct((N, H), F32),
            jax.ShapeDtypeStruct((N, 1), F32),
        ],
    )(x, lin1_wt, lin1_b, w1x_t, lin2_t, att_r)


def _tc_edge_v(ea_pad, w1e_t):
    Epad = ea_pad.shape[0]
    BE = 4096

    def body(ea_ref, w_ref, v_ref):
        v_ref[...] = _dot(ea_ref[...], w_ref[...])

    return pl.pallas_call(
        body,
        grid=(Epad // BE,),
        in_specs=[
            pl.BlockSpec((BE, ea_pad.shape[1]), lambda i: (i, 0)),
            pl.BlockSpec(w1e_t.shape, lambda i: (0, 0)),
        ],
        out_specs=pl.BlockSpec((BE, H), lambda i: (i, 0)),
        out_shape=jax.ShapeDtypeStruct((Epad, H), F32),
    )(ea_pad, w1e_t)


def _tc_reduce(parts):
    _, NR, _ = parts.shape

    def body(p_ref, o_ref):
        o_ref[...] = jnp.sum(p_ref[...], axis=0)

    return pl.pallas_call(
        body,
        out_shape=jax.ShapeDtypeStruct((NR, 128), F32),
    )(parts)


def _tc_block(hacc, ssum, bias, xprev, wih, whh, bih, bhh,
              wnext_t, asrc, adst):
    N = xprev.shape[0]

    def body(hacc_ref, ss_ref, b_ref, xp_ref, wih_ref, whh_ref,
             bih_ref, bhh_ref, wn_ref, as_ref, ad_ref,
             xn_ref, xs_ref, s_ref, d_ref):
        h = _elu(hacc_ref[...] / (ss_ref[...] + 1e-16) + b_ref[...])
        xnew = jax.nn.relu(_gru(h, xp_ref[...], wih_ref[...], whh_ref[...],
                                bih_ref[...], bhh_ref[...]))
        xn_ref[...] = xnew
        xs = _dot(xnew, wn_ref[...])
        xs_ref[...] = xs
        s_ref[...] = _dot(xs, as_ref[...])
        d_ref[...] = _dot(xs, ad_ref[...])

    return pl.pallas_call(
        body,
        out_shape=[
            jax.ShapeDtypeStruct((N, H), F32),
            jax.ShapeDtypeStruct((N, H), F32),
            jax.ShapeDtypeStruct((N, 1), F32),
            jax.ShapeDtypeStruct((N, 1), F32),
        ],
    )(hacc, ssum, bias, xprev, wih, whh, bih, bhh, wnext_t, asrc, adst)


def _tc_mol(hacc, ssum, bias2, xprev, wih2, whh2, bih2, bhh2,
            batch_col, mol_wt, m_asrc, m_adst, mol_bias,
            mg_wih, mg_whh, mg_bih, mg_bhh, lin2_t, lin2_b, out_t, out_b):
    N = xprev.shape[0]
    G = 64
    OUT = out_t.shape[1]

    def body(hacc_ref, ss_ref, b2_ref, xp_ref, wih_ref, whh_ref,
             bih_ref, bhh_ref, bc_ref, mw_ref, mas_ref, mad_ref, mb_ref,
             gwih_ref, gwhh_ref, gbih_ref, gbhh_ref,
             l2_ref, l2b_ref, ow_ref, ob_ref, res_ref):
        h = _elu(hacc_ref[...] / (ss_ref[...] + 1e-16) + b2_ref[...])
        x4 = jax.nn.relu(_gru(h, xp_ref[...], wih_ref[...], whh_ref[...],
                              bih_ref[...], bhh_ref[...]))
        gid = lax.broadcasted_iota(jnp.int32, (N, G), 1)
        B = jnp.where(bc_ref[...] == gid, 1.0, 0.0).astype(F32)
        out = jax.nn.relu(_dotT(B, x4))
        xs = _dot(x4, mw_ref[...])
        s = _dot(xs, mas_ref[...])
        for _ in range(2):
            d = _dot(_dot(out, mw_ref[...]), mad_ref[...])
            dn = _dot(B, d)
            e = jnp.exp(_lk(s + dn))
            sg = _dotT(B, e)
            hm = _dotT(B, e * xs)
            hg = _elu(hm / (sg + 1e-16) + mb_ref[...])
            out = jax.nn.relu(_gru(hg, out, gwih_ref[...], gwhh_ref[...],
                                   gbih_ref[...], gbhh_ref[...]))
        r2 = _dot(out, l2_ref[...]) + l2b_ref[...]
        res_ref[...] = _dot(r2, ow_ref[...]) + ob_ref[...]

    return pl.pallas_call(
        body,
        out_shape=jax.ShapeDtypeStruct((G, OUT), F32),
    )(hacc, ssum, bias2, xprev, wih2, whh2, bih2, bhh2, batch_col,
      mol_wt, m_asrc, m_adst, mol_bias, mg_wih, mg_whh, mg_bih, mg_bhh,
      lin2_t, lin2_b, out_t, out_b)


def _mesh():
    return plsc.VectorSubcoreMesh(core_axis_name="c", subcore_axis_name="s")


def _sc_gat_scalar(sv2, dv2, src3, dst3, znr, E):
    NR = sv2.shape[0]
    NCH = src3.shape[1]
    EPT = NCH * 128

    @functools.partial(
        pl.kernel, mesh=_mesh(),
        compiler_params=pltpu.CompilerParams(needs_layout_passes=False),
        out_type=[
            jax.ShapeDtypeStruct((NW, NCH, 128), F32),
            jax.ShapeDtypeStruct((NW, NR, 128), F32),
        ],
        scratch_types=[
            pltpu.VMEM((NR, 128), F32),
            pltpu.VMEM((NR, 128), F32),
            pltpu.VMEM((NCH, 128), jnp.int32),
            pltpu.VMEM((NCH, 128), jnp.int32),
            pltpu.VMEM((NCH, 128), F32),
            pltpu.VMEM((NR, 128), F32),
            pltpu.SemaphoreType.DMA,
        ],
    )
    def k(sv_hbm, dv_hbm, src_hbm, dst_hbm, znr_hbm,
          e_out, ssum_out,
          sv_v, dv_v, src_v, dst_v, e_t, ssum_t, sem):
        c = lax.axis_index("c")
        s = lax.axis_index("s")
        wid = c * NS + s
        pltpu.sync_copy(sv_hbm, sv_v)
        pltpu.sync_copy(dv_hbm, dv_v)
        pltpu.sync_copy(src_hbm.at[wid], src_v)
        pltpu.sync_copy(dst_hbm.at[wid], dst_v)
        pltpu.sync_copy(znr_hbm, ssum_t)
        base = wid * EPT
        lane = lax.broadcasted_iota(jnp.int32, (16,), 0)

        def group(j, g, _):
            sl = pl.ds(g * 16, 16)
            src16 = src_v[j, sl]
            dst16 = dst_v[j, sl]
            s16 = plsc.load_gather(
                sv_v, [lax.shift_right_logical(src16, 7),
                       jnp.bitwise_and(src16, 127)])
            d16 = plsc.load_gather(
                dv_v, [lax.shift_right_logical(dst16, 7),
                       jnp.bitwise_and(dst16, 127)])
            e16 = jnp.exp(_lk(s16 + d16))
            eid = base + j * 128 + g * 16 + lane
            e16 = jnp.where(eid < E, e16, 0.0)
            e_t[j, sl] = e16
            plsc.addupdate_scatter(
                ssum_t, [lax.shift_right_logical(dst16, 7),
                         jnp.bitwise_and(dst16, 127)], e16)
            return 0

        def chunk(j, _):
            lax.fori_loop(0, 8, functools.partial(group, j), 0)
            return 0

        lax.fori_loop(0, NCH, chunk, 0)
        pltpu.sync_copy(e_t, e_out.at[wid])
        pltpu.sync_copy(ssum_t, ssum_out.at[wid])

    return k(sv2, dv2, src3, dst3, znr)


def _sc_gate_scalar(u, rv2, attl, vpad, src3, dst3, znr, E):
    NR = rv2.shape[0]
    NCH = src3.shape[1]
    EPT = NCH * 128

    @functools.partial(
        pl.kernel, mesh=_mesh(),
        compiler_params=pltpu.CompilerParams(needs_layout_passes=False),
        out_type=[
            jax.ShapeDtypeStruct((NW, NCH, 128), F32),
            jax.ShapeDtypeStruct((NW, NR, 128), F32),
        ],
        scratch_types=[
            pltpu.VMEM((NR, 128), F32),
            pltpu.VMEM((8, 16), F32),
            pltpu.VMEM((NCH, 128), jnp.int32),
            pltpu.VMEM((NCH, 128), jnp.int32),
            pltpu.VMEM((128, H), F32),
            pltpu.VMEM((128, H), F32),
            pltpu.VMEM((NCH, 128), F32),
            pltpu.VMEM((NR, 128), F32),
            pltpu.SemaphoreType.DMA,
        ],
    )
    def k(u_hbm, rv_hbm, al_hbm, v_hbm, src_hbm, dst_hbm, znr_hbm,
          e_out, ssum_out,
          rv_v, al_v, src_v, dst_v, ru_v, v_v, e_t, ssum_t, sem):
        c = lax.axis_index("c")
        s = lax.axis_index("s")
        wid = c * NS + s
        pltpu.sync_copy(rv_hbm, rv_v)
        pltpu.sync_copy(al_hbm, al_v)
        pltpu.sync_copy(src_hbm.at[wid], src_v)
        pltpu.sync_copy(dst_hbm.at[wid], dst_v)
        pltpu.sync_copy(znr_hbm, ssum_t)
        base = wid * EPT
        lane = lax.broadcasted_iota(jnp.int32, (16,), 0)
        alc = [al_v[kk, :] for kk in range(8)]

        def group(j, g, _):
            sl = pl.ds(g * 16, 16)
            dst16 = dst_v[j, sl]
            r16 = plsc.load_gather(
                rv_v, [lax.shift_right_logical(dst16, 7),
                       jnp.bitwise_and(dst16, 127)])
            ea16 = jnp.zeros((16,), F32)
            for i in range(16):
                ri = g * 16 + i
                acc = jnp.zeros((16,), F32)
                for kk in range(8):
                    ks = pl.ds(kk * 16, 16)
                    acc = acc + _lk(ru_v[ri, ks] + v_v[ri, ks]) * alc[kk]
                ea16 = jnp.where(lane == i, jnp.sum(acc), ea16)
            e16 = jnp.exp(_lk(ea16 + r16))
            eid = base + j * 128 + g * 16 + lane
            e16 = jnp.where(eid < E, e16, 0.0)
            e_t[j, sl] = e16
            plsc.addupdate_scatter(
                ssum_t, [lax.shift_right_logical(dst16, 7),
                         jnp.bitwise_and(dst16, 127)], e16)
            return 0

        def chunk(j, _):
            cp1 = pltpu.async_copy(u_hbm.at[src_v.at[j]], ru_v, sem)
            cp2 = pltpu.async_copy(v_hbm.at[pl.ds(base + j * 128, 128)],
                                   v_v, sem)
            cp1.wait()
            cp2.wait()
            lax.fori_loop(0, 8, functools.partial(group, j), 0)
            return 0

        lax.fori_loop(0, NCH, chunk, 0)
        pltpu.sync_copy(e_t, e_out.at[wid])
        pltpu.sync_copy(ssum_t, ssum_out.at[wid])

    return k(u, rv2, attl, vpad, src3, dst3, znr)


def _sc_rows(mtab, srcf, dstf, ef, zn4):
    NFR = mtab.shape[1]
    NR = NFR // FPT
    NCHA = srcf.shape[0]
    NSUP = NCHA // 16

    @functools.partial(
        pl.kernel, mesh=_mesh(),
        compiler_params=pltpu.CompilerParams(needs_layout_passes=False),
        out_type=jax.ShapeDtypeStruct((NW, NFR, 128), F32),
        scratch_types=[
            pltpu.VMEM((NFR, 128), F32),
            pltpu.VMEM((NFR, 128), F32),
            pltpu.VMEM((16, 128), jnp.int32),
            pltpu.VMEM((16, 128), jnp.int32),
            pltpu.VMEM((16, 128), F32),
            pltpu.SemaphoreType.DMA,
        ],
    )
    def k(mt_hbm, src_hbm, dst_hbm, e_hbm, zn4_hbm,
          acc_out,
          m4_v, acc_v, sbuf, dbuf, ebuf, sem):
        c = lax.axis_index("c")
        s = lax.axis_index("s")
        wid = c * NS + s
        pltpu.sync_copy(mt_hbm.at[wid], m4_v)
        pltpu.sync_copy(zn4_hbm, acc_v)

        def sup(t, _):
            sl16 = pl.ds(t * 16, 16)
            pltpu.sync_copy(src_hbm.at[sl16], sbuf)
            pltpu.sync_copy(dst_hbm.at[sl16], dbuf)
            pltpu.sync_copy(e_hbm.at[sl16], ebuf)
            for r in range(16):
                for g in range(8):
                    sl = pl.ds(g * 16, 16)
                    src16 = sbuf[r, sl]
                    dst16 = dbuf[r, sl]
                    e16 = ebuf[r, sl]
                    sr = lax.shift_right_logical(src16, 7)
                    scol = jnp.bitwise_and(src16, 127)
                    dr = lax.shift_right_logical(dst16, 7)
                    dcol = jnp.bitwise_and(dst16, 127)
                    for fi in range(FPT):
                        v = plsc.load_gather(m4_v, [sr + (fi * NR), scol])
                        plsc.addupdate_scatter(acc_v,
                                               [dr + (fi * NR), dcol],
                                               v * e16)
            return 0

        lax.fori_loop(0, NSUP, sup, 0)
        pltpu.sync_copy(acc_v, acc_out.at[wid])

    return k(mtab, srcf, dstf, ef, zn4)


def kernel(x, edge_index, edge_attr, batch, params):
    p = params
    N = x.shape[0]
    E = edge_index.shape[1]
    DE = edge_attr.shape[1]

    EPT = ((E + NW * 128 - 1) // (NW * 128)) * 128
    Epad = EPT * NW
    NCH = EPT // 128
    NCHA = Epad // 128
    src3 = jnp.concatenate([edge_index[0],
                            jnp.zeros((Epad - E,), jnp.int32)]).reshape(NW, NCH, 128)
    dst3 = jnp.concatenate([edge_index[1],
                            jnp.zeros((Epad - E,), jnp.int32)]).reshape(NW, NCH, 128)
    srcf = src3.reshape(NCHA, 128)
    dstf = dst3.reshape(NCHA, 128)
    ea_pad = jnp.concatenate(
        [edge_attr, jnp.zeros((Epad - E, DE), F32)], axis=0)

    NR = (N + 127) // 128
    NP = NR * 128
    znr = jnp.zeros((NR, 128), F32)
    zn4 = jnp.zeros((FPT * NR, 128), F32)
    padN = lambda v: jnp.concatenate(
        [v, jnp.zeros((NP - N,), F32)]).reshape(NR, 128)

    def ftab(t):
        tt = jnp.pad(t.T, ((0, 0), (0, NP - N)))
        return tt.reshape(NW, FPT * NR, 128)

    r2 = lambda v: v.reshape(1, -1)
    c2 = lambda v: v.reshape(-1, 1)

    def finish(acc, parts):
        hacc = acc.reshape(H, NP)[:, :N].T
        ssum = _tc_reduce(parts).reshape(NP)[:N].reshape(N, 1)
        return hacc, ssum

    x1, u, m, r = _tc_node_pre(
        x, p['lin1_W'].T, r2(p['lin1_b']),
        p['gate_lin1_W'][:, :H].T, p['gate_lin2_W'].T, c2(p['gate_att_r']))

    vpad = _tc_edge_v(ea_pad, p['gate_lin1_W'][:, H:].T)

    ef, parts = _sc_gate_scalar(u, padN(r.reshape(-1)),
                                p['gate_att_l'].reshape(8, 16), vpad,
                                src3, dst3, znr, E)
    acc = _sc_rows(ftab(m), srcf, dstf, ef.reshape(NCHA, 128), zn4)
    hacc, ssum = finish(acc, parts)

    xcur = x1
    for i, (wih, whh, bih, bhh, bias_in) in enumerate([
            (p['gru0_Wih'], p['gru0_Whh'], p['gru0_bih'], p['gru0_bhh'],
             r2(p['gate_bias'])),
            (p['gru1_Wih'], p['gru1_Whh'], p['gru1_bih'], p['gru1_bhh'],
             r2(p['conv1_bias'])),
    ]):
        li = i + 1
        xcur, xs, sv, dv = _tc_block(
            hacc, ssum, bias_in, xcur, wih.T, whh.T, r2(bih), r2(bhh),
            p['conv%d_lin_W' % li].T, c2(p['conv%d_att_src' % li]),
            c2(p['conv%d_att_dst' % li]))
        ef, parts = _sc_gat_scalar(padN(sv.reshape(-1)), padN(dv.reshape(-1)),
                                   src3, dst3, znr, E)
        acc = _sc_rows(ftab(xs), srcf, dstf, ef.reshape(NCHA, 128), zn4)
        hacc, ssum = finish(acc, parts)

    out = _tc_mol(
        hacc, ssum, r2(p['conv2_bias']), xcur,
        p['gru2_Wih'].T, p['gru2_Whh'].T, r2(p['gru2_bih']), r2(p['gru2_bhh']),
        batch.reshape(-1, 1), p['mol_lin_W'].T, c2(p['mol_att_src']),
        c2(p['mol_att_dst']), r2(p['mol_bias']),
        p['molgru_Wih'].T, p['molgru_Whh'].T, r2(p['molgru_bih']),
        r2(p['molgru_bhh']),
        p['lin2_W'].T, r2(p['lin2_b']), p['out_W'].T, r2(p['out_b']))
    return out


# speedup vs baseline: 1.1955x; 1.1955x over previous
import functools

import jax
import jax.numpy as jnp
from jax import lax
from jax.experimental import pallas as pl
from jax.experimental.pallas import tpu as pltpu
from jax.experimental.pallas import tpu_sc as plsc

F32 = jnp.float32
H = 128
NC = 2
NS = 16
NW = NC * NS
FPT = H // NW


def _lk(x):
    return jnp.maximum(x, 0.01 * x)


def _elu(x):
    return jnp.where(x > 0, x, jnp.exp(x) - 1.0)


def _sig(x):
    return 1.0 / (1.0 + jnp.exp(-x))


def _dot(a, b):
    return jnp.dot(a, b, preferred_element_type=F32)


def _dotT(a, b):
    return lax.dot_general(a, b, (((0,), (0,)), ((), ())),
                           preferred_element_type=F32)


def _gru(inp, hid, wih, whh, bih, bhh):
    gi = _dot(inp, wih) + bih
    gh = _dot(hid, whh) + bhh
    r = _sig(gi[:, :H] + gh[:, :H])
    z = _sig(gi[:, H:2 * H] + gh[:, H:2 * H])
    n = jnp.tanh(gi[:, 2 * H:] + r * gh[:, 2 * H:])
    return (1.0 - z) * n + z * hid


def _tc_node_pre(x, lin1_wt, lin1_b, w1x_t, lin2_t, att_r):
    N = x.shape[0]

    def body(x_ref, w1_ref, b1_ref, wx_ref, w2_ref, ar_ref,
             x1_ref, u_ref, m_ref, r_ref):
        x1 = _lk(_dot(x_ref[...], w1_ref[...]) + b1_ref[...])
        x1_ref[...] = x1
        u_ref[...] = _dot(x1, wx_ref[...])
        m_ref[...] = _dot(x1, w2_ref[...])
        r_ref[...] = _dot(x1, ar_ref[...])

    return pl.pallas_call(
        body,
        out_shape=[
            jax.ShapeDtypeStruct((N, H), F32),
            jax.ShapeDtypeStruct((N, H), F32),
            jax.ShapeDtypeStruct((N, H), F32),
            jax.ShapeDtypeStruct((N, 1), F32),
        ],
    )(x, lin1_wt, lin1_b, w1x_t, lin2_t, att_r)


def _tc_edge_v(ea_pad, w1e_t):
    Epad = ea_pad.shape[0]
    BE = 4096

    def body(ea_ref, w_ref, v_ref):
        v_ref[...] = _dot(ea_ref[...], w_ref[...])

    return pl.pallas_call(
        body,
        grid=(Epad // BE,),
        in_specs=[
            pl.BlockSpec((BE, ea_pad.shape[1]), lambda i: (i, 0)),
            pl.BlockSpec(w1e_t.shape, lambda i: (0, 0)),
        ],
        out_specs=pl.BlockSpec((BE, H), lambda i: (i, 0)),
        out_shape=jax.ShapeDtypeStruct((Epad, H), F32),
    )(ea_pad, w1e_t)


def _tc_reduce(parts):
    _, NR, _ = parts.shape

    def body(p_ref, o_ref):
        o_ref[...] = jnp.sum(p_ref[...], axis=0)

    return pl.pallas_call(
        body,
        out_shape=jax.ShapeDtypeStruct((NR, 128), F32),
    )(parts)


def _tc_block(hacc, ssum, bias, xprev, wih, whh, bih, bhh,
              wnext_t, asrc, adst):
    N = xprev.shape[0]

    def body(hacc_ref, ss_ref, b_ref, xp_ref, wih_ref, whh_ref,
             bih_ref, bhh_ref, wn_ref, as_ref, ad_ref,
             xn_ref, xs_ref, s_ref, d_ref):
        h = _elu(hacc_ref[...] / (ss_ref[...] + 1e-16) + b_ref[...])
        xnew = jax.nn.relu(_gru(h, xp_ref[...], wih_ref[...], whh_ref[...],
                                bih_ref[...], bhh_ref[...]))
        xn_ref[...] = xnew
        xs = _dot(xnew, wn_ref[...])
        xs_ref[...] = xs
        s_ref[...] = _dot(xs, as_ref[...])
        d_ref[...] = _dot(xs, ad_ref[...])

    return pl.pallas_call(
        body,
        out_shape=[
            jax.ShapeDtypeStruct((N, H), F32),
            jax.ShapeDtypeStruct((N, H), F32),
            jax.ShapeDtypeStruct((N, 1), F32),
            jax.ShapeDtypeStruct((N, 1), F32),
        ],
    )(hacc, ssum, bias, xprev, wih, whh, bih, bhh, wnext_t, asrc, adst)


def _tc_mol(hacc, ssum, bias2, xprev, wih2, whh2, bih2, bhh2,
            batch_col, mol_wt, m_asrc, m_adst, mol_bias,
            mg_wih, mg_whh, mg_bih, mg_bhh, lin2_t, lin2_b, out_t, out_b):
    N = xprev.shape[0]
    G = 64
    OUT = out_t.shape[1]

    def body(hacc_ref, ss_ref, b2_ref, xp_ref, wih_ref, whh_ref,
             bih_ref, bhh_ref, bc_ref, mw_ref, mas_ref, mad_ref, mb_ref,
             gwih_ref, gwhh_ref, gbih_ref, gbhh_ref,
             l2_ref, l2b_ref, ow_ref, ob_ref, res_ref):
        h = _elu(hacc_ref[...] / (ss_ref[...] + 1e-16) + b2_ref[...])
        x4 = jax.nn.relu(_gru(h, xp_ref[...], wih_ref[...], whh_ref[...],
                              bih_ref[...], bhh_ref[...]))
        gid = lax.broadcasted_iota(jnp.int32, (N, G), 1)
        B = jnp.where(bc_ref[...] == gid, 1.0, 0.0).astype(F32)
        out = jax.nn.relu(_dotT(B, x4))
        xs = _dot(x4, mw_ref[...])
        s = _dot(xs, mas_ref[...])
        for _ in range(2):
            d = _dot(_dot(out, mw_ref[...]), mad_ref[...])
            dn = _dot(B, d)
            e = jnp.exp(_lk(s + dn))
            sg = _dotT(B, e)
            hm = _dotT(B, e * xs)
            hg = _elu(hm / (sg + 1e-16) + mb_ref[...])
            out = jax.nn.relu(_gru(hg, out, gwih_ref[...], gwhh_ref[...],
                                   gbih_ref[...], gbhh_ref[...]))
        r2 = _dot(out, l2_ref[...]) + l2b_ref[...]
        res_ref[...] = _dot(r2, ow_ref[...]) + ob_ref[...]

    return pl.pallas_call(
        body,
        out_shape=jax.ShapeDtypeStruct((G, OUT), F32),
    )(hacc, ssum, bias2, xprev, wih2, whh2, bih2, bhh2, batch_col,
      mol_wt, m_asrc, m_adst, mol_bias, mg_wih, mg_whh, mg_bih, mg_bhh,
      lin2_t, lin2_b, out_t, out_b)


def _mesh():
    return plsc.VectorSubcoreMesh(core_axis_name="c", subcore_axis_name="s")


def _sc_gat_scalar(sv2, dv2, src3, dst3, znr, E):
    NR = sv2.shape[0]
    NCH = src3.shape[1]
    EPT = NCH * 128

    @functools.partial(
        pl.kernel, mesh=_mesh(),
        compiler_params=pltpu.CompilerParams(needs_layout_passes=False),
        out_type=[
            jax.ShapeDtypeStruct((NW, NCH, 128), F32),
            jax.ShapeDtypeStruct((NW, NR, 128), F32),
        ],
        scratch_types=[
            pltpu.VMEM((NR, 128), F32),
            pltpu.VMEM((NR, 128), F32),
            pltpu.VMEM((NCH, 128), jnp.int32),
            pltpu.VMEM((NCH, 128), jnp.int32),
            pltpu.VMEM((NCH, 128), F32),
            pltpu.VMEM((NR, 128), F32),
            pltpu.SemaphoreType.DMA,
        ],
    )
    def k(sv_hbm, dv_hbm, src_hbm, dst_hbm, znr_hbm,
          e_out, ssum_out,
          sv_v, dv_v, src_v, dst_v, e_t, ssum_t, sem):
        c = lax.axis_index("c")
        s = lax.axis_index("s")
        wid = c * NS + s
        pltpu.sync_copy(sv_hbm, sv_v)
        pltpu.sync_copy(dv_hbm, dv_v)
        pltpu.sync_copy(src_hbm.at[wid], src_v)
        pltpu.sync_copy(dst_hbm.at[wid], dst_v)
        pltpu.sync_copy(znr_hbm, ssum_t)
        base = wid * EPT
        lane = lax.broadcasted_iota(jnp.int32, (16,), 0)

        def group(j, g, _):
            sl = pl.ds(g * 16, 16)
            src16 = src_v[j, sl]
            dst16 = dst_v[j, sl]
            s16 = plsc.load_gather(
                sv_v, [lax.shift_right_logical(src16, 7),
                       jnp.bitwise_and(src16, 127)])
            d16 = plsc.load_gather(
                dv_v, [lax.shift_right_logical(dst16, 7),
                       jnp.bitwise_and(dst16, 127)])
            e16 = jnp.exp(_lk(s16 + d16))
            eid = base + j * 128 + g * 16 + lane
            e16 = jnp.where(eid < E, e16, 0.0)
            e_t[j, sl] = e16
            plsc.addupdate_scatter(
                ssum_t, [lax.shift_right_logical(dst16, 7),
                         jnp.bitwise_and(dst16, 127)], e16)
            return 0

        def chunk(j, _):
            lax.fori_loop(0, 8, functools.partial(group, j), 0)
            return 0

        lax.fori_loop(0, NCH, chunk, 0)
        pltpu.sync_copy(e_t, e_out.at[wid])
        pltpu.sync_copy(ssum_t, ssum_out.at[wid])

    return k(sv2, dv2, src3, dst3, znr)


def _sc_gate_scalar(u, rv2, attl, vpad, src3, dst3, znr, E):
    NR = rv2.shape[0]
    NCH = src3.shape[1]
    EPT = NCH * 128

    @functools.partial(
        pl.kernel, mesh=_mesh(),
        compiler_params=pltpu.CompilerParams(needs_layout_passes=False),
        out_type=[
            jax.ShapeDtypeStruct((NW, NCH, 128), F32),
            jax.ShapeDtypeStruct((NW, NR, 128), F32),
        ],
        scratch_types=[
            pltpu.VMEM((NR, 128), F32),
            pltpu.VMEM((8, 16), F32),
            pltpu.VMEM((NCH, 128), jnp.int32),
            pltpu.VMEM((NCH, 128), jnp.int32),
            pltpu.VMEM((128, H), F32),
            pltpu.VMEM((128, H), F32),
            pltpu.VMEM((NCH, 128), F32),
            pltpu.VMEM((NR, 128), F32),
            pltpu.SemaphoreType.DMA,
        ],
    )
    def k(u_hbm, rv_hbm, al_hbm, v_hbm, src_hbm, dst_hbm, znr_hbm,
          e_out, ssum_out,
          rv_v, al_v, src_v, dst_v, ru_v, v_v, e_t, ssum_t, sem):
        c = lax.axis_index("c")
        s = lax.axis_index("s")
        wid = c * NS + s
        pltpu.sync_copy(rv_hbm, rv_v)
        pltpu.sync_copy(al_hbm, al_v)
        pltpu.sync_copy(src_hbm.at[wid], src_v)
        pltpu.sync_copy(dst_hbm.at[wid], dst_v)
        pltpu.sync_copy(znr_hbm, ssum_t)
        base = wid * EPT
        lane = lax.broadcasted_iota(jnp.int32, (16,), 0)
        alc = [al_v[kk, :] for kk in range(8)]

        def group(j, g, _):
            sl = pl.ds(g * 16, 16)
            dst16 = dst_v[j, sl]
            r16 = plsc.load_gather(
                rv_v, [lax.shift_right_logical(dst16, 7),
                       jnp.bitwise_and(dst16, 127)])
            ea16 = jnp.zeros((16,), F32)
            for i in range(16):
                ri = g * 16 + i
                acc = jnp.zeros((16,), F32)
                for kk in range(8):
                    ks = pl.ds(kk * 16, 16)
                    acc = acc + _lk(ru_v[ri, ks] + v_v[ri, ks]) * alc[kk]
                ea16 = jnp.where(lane == i, jnp.sum(acc), ea16)
            e16 = jnp.exp(_lk(ea16 + r16))
            eid = base + j * 128 + g * 16 + lane
            e16 = jnp.where(eid < E, e16, 0.0)
            e_t[j, sl] = e16
            plsc.addupdate_scatter(
                ssum_t, [lax.shift_right_logical(dst16, 7),
                         jnp.bitwise_and(dst16, 127)], e16)
            return 0

        def chunk(j, _):
            cp1 = pltpu.async_copy(u_hbm.at[src_v.at[j]], ru_v, sem)
            cp2 = pltpu.async_copy(v_hbm.at[pl.ds(base + j * 128, 128)],
                                   v_v, sem)
            cp1.wait()
            cp2.wait()
            lax.fori_loop(0, 8, functools.partial(group, j), 0)
            return 0

        lax.fori_loop(0, NCH, chunk, 0)
        pltpu.sync_copy(e_t, e_out.at[wid])
        pltpu.sync_copy(ssum_t, ssum_out.at[wid])

    return k(u, rv2, attl, vpad, src3, dst3, znr)


def _sc_rows(mtab, srcf, dstf, ef, zn4):
    NFR = mtab.shape[1]
    NR = NFR // FPT
    NCHA = srcf.shape[0]
    NSUP = NCHA // 16

    @functools.partial(
        pl.kernel, mesh=_mesh(),
        compiler_params=pltpu.CompilerParams(needs_layout_passes=False),
        out_type=jax.ShapeDtypeStruct((NW, NFR, 128), F32),
        scratch_types=[
            pltpu.VMEM((NFR, 128), F32),
            pltpu.VMEM((NFR, 128), F32),
            pltpu.VMEM((16, 128), jnp.int32),
            pltpu.VMEM((16, 128), jnp.int32),
            pltpu.VMEM((16, 128), F32),
            pltpu.SemaphoreType.DMA,
        ],
    )
    def k(mt_hbm, src_hbm, dst_hbm, e_hbm, zn4_hbm,
          acc_out,
          m4_v, acc_v, sbuf, dbuf, ebuf, sem):
        c = lax.axis_index("c")
        s = lax.axis_index("s")
        wid = c * NS + s
        pltpu.sync_copy(mt_hbm.at[wid], m4_v)
        pltpu.sync_copy(zn4_hbm, acc_v)

        def row(r, _):
            for g in range(8):
                sl = pl.ds(g * 16, 16)
                src16 = sbuf[r, sl]
                dst16 = dbuf[r, sl]
                e16 = ebuf[r, sl]
                sr = lax.shift_right_logical(src16, 7)
                scol = jnp.bitwise_and(src16, 127)
                dr = lax.shift_right_logical(dst16, 7)
                dcol = jnp.bitwise_and(dst16, 127)
                for fi in range(FPT):
                    v = plsc.load_gather(m4_v, [sr + (fi * NR), scol])
                    plsc.addupdate_scatter(acc_v,
                                           [dr + (fi * NR), dcol],
                                           v * e16)
            return 0

        def sup(t, _):
            sl16 = pl.ds(t * 16, 16)
            pltpu.sync_copy(src_hbm.at[sl16], sbuf)
            pltpu.sync_copy(dst_hbm.at[sl16], dbuf)
            pltpu.sync_copy(e_hbm.at[sl16], ebuf)
            lax.fori_loop(0, 16, row, 0)
            return 0

        lax.fori_loop(0, NSUP, sup, 0)
        pltpu.sync_copy(acc_v, acc_out.at[wid])

    return k(mtab, srcf, dstf, ef, zn4)


def kernel(x, edge_index, edge_attr, batch, params):
    p = params
    N = x.shape[0]
    E = edge_index.shape[1]
    DE = edge_attr.shape[1]

    EPT = ((E + NW * 128 - 1) // (NW * 128)) * 128
    Epad = EPT * NW
    NCH = EPT // 128
    NCHA = Epad // 128
    src3 = jnp.concatenate([edge_index[0],
                            jnp.zeros((Epad - E,), jnp.int32)]).reshape(NW, NCH, 128)
    dst3 = jnp.concatenate([edge_index[1],
                            jnp.zeros((Epad - E,), jnp.int32)]).reshape(NW, NCH, 128)
    srcf = src3.reshape(NCHA, 128)
    dstf = dst3.reshape(NCHA, 128)
    ea_pad = jnp.concatenate(
        [edge_attr, jnp.zeros((Epad - E, DE), F32)], axis=0)

    NR = (N + 127) // 128
    NP = NR * 128
    znr = jnp.zeros((NR, 128), F32)
    zn4 = jnp.zeros((FPT * NR, 128), F32)
    padN = lambda v: jnp.concatenate(
        [v, jnp.zeros((NP - N,), F32)]).reshape(NR, 128)

    def ftab(t):
        tt = jnp.pad(t.T, ((0, 0), (0, NP - N)))
        return tt.reshape(NW, FPT * NR, 128)

    r2 = lambda v: v.reshape(1, -1)
    c2 = lambda v: v.reshape(-1, 1)

    def finish(acc, parts):
        hacc = acc.reshape(H, NP)[:, :N].T
        ssum = _tc_reduce(parts).reshape(NP)[:N].reshape(N, 1)
        return hacc, ssum

    x1, u, m, r = _tc_node_pre(
        x, p['lin1_W'].T, r2(p['lin1_b']),
        p['gate_lin1_W'][:, :H].T, p['gate_lin2_W'].T, c2(p['gate_att_r']))

    vpad = _tc_edge_v(ea_pad, p['gate_lin1_W'][:, H:].T)

    ef, parts = _sc_gate_scalar(u, padN(r.reshape(-1)),
                                p['gate_att_l'].reshape(8, 16), vpad,
                                src3, dst3, znr, E)
    acc = _sc_rows(ftab(m), srcf, dstf, ef.reshape(NCHA, 128), zn4)
    hacc, ssum = finish(acc, parts)

    xcur = x1
    for i, (wih, whh, bih, bhh, bias_in) in enumerate([
            (p['gru0_Wih'], p['gru0_Whh'], p['gru0_bih'], p['gru0_bhh'],
             r2(p['gate_bias'])),
            (p['gru1_Wih'], p['gru1_Whh'], p['gru1_bih'], p['gru1_bhh'],
             r2(p['conv1_bias'])),
    ]):
        li = i + 1
        xcur, xs, sv, dv = _tc_block(
            hacc, ssum, bias_in, xcur, wih.T, whh.T, r2(bih), r2(bhh),
            p['conv%d_lin_W' % li].T, c2(p['conv%d_att_src' % li]),
            c2(p['conv%d_att_dst' % li]))
        ef, parts = _sc_gat_scalar(padN(sv.reshape(-1)), padN(dv.reshape(-1)),
                                   src3, dst3, znr, E)
        acc = _sc_rows(ftab(xs), srcf, dstf, ef.reshape(NCHA, 128), zn4)
        hacc, ssum = finish(acc, parts)

    out = _tc_mol(
        hacc, ssum, r2(p['conv2_bias']), xcur,
        p['gru2_Wih'].T, p['gru2_Whh'].T, r2(p['gru2_bih']), r2(p['gru2_bhh']),
        batch.reshape(-1, 1), p['mol_lin_W'].T, c2(p['mol_att_src']),
        c2(p['mol_att_dst']), r2(p['mol_bias']),
        p['molgru_Wih'].T, p['molgru_Whh'].T, r2(p['molgru_bih']),
        r2(p['molgru_bhh']),
        p['lin2_W'].T, r2(p['lin2_b']), p['out_W'].T, r2(p['out_b']))
    return out


# speedup vs baseline: 1.7155x; 1.4349x over previous
import functools

import jax
import jax.numpy as jnp
from jax import lax
from jax.experimental import pallas as pl
from jax.experimental.pallas import tpu as pltpu
from jax.experimental.pallas import tpu_sc as plsc

F32 = jnp.float32
H = 128
NC = 2
NS = 16
NW = NC * NS
FPT = H // NW


def _lk(x):
    return jnp.maximum(x, 0.01 * x)


def _elu(x):
    return jnp.where(x > 0, x, jnp.exp(x) - 1.0)


def _sig(x):
    return 1.0 / (1.0 + jnp.exp(-x))


def _dot(a, b):
    return jnp.dot(a, b, preferred_element_type=F32)


def _dotT(a, b):
    return lax.dot_general(a, b, (((0,), (0,)), ((), ())),
                           preferred_element_type=F32)


def _gru(inp, hid, wih, whh, bih, bhh):
    gi = _dot(inp, wih) + bih
    gh = _dot(hid, whh) + bhh
    r = _sig(gi[:, :H] + gh[:, :H])
    z = _sig(gi[:, H:2 * H] + gh[:, H:2 * H])
    n = jnp.tanh(gi[:, 2 * H:] + r * gh[:, 2 * H:])
    return (1.0 - z) * n + z * hid


def _tc_node_pre(x, lin1_wt, lin1_b, w1x_t, lin2_t, att_r):
    N = x.shape[0]

    def body(x_ref, w1_ref, b1_ref, wx_ref, w2_ref, ar_ref,
             x1_ref, u_ref, m_ref, r_ref):
        x1 = _lk(_dot(x_ref[...], w1_ref[...]) + b1_ref[...])
        x1_ref[...] = x1
        u_ref[...] = _dot(x1, wx_ref[...])
        m_ref[...] = _dot(x1, w2_ref[...])
        r_ref[...] = _dot(x1, ar_ref[...])

    return pl.pallas_call(
        body,
        out_shape=[
            jax.ShapeDtypeStruct((N, H), F32),
            jax.ShapeDtypeStruct((N, H), F32),
            jax.ShapeDtypeStruct((N, H), F32),
            jax.ShapeDtypeStruct((N, 1), F32),
        ],
    )(x, lin1_wt, lin1_b, w1x_t, lin2_t, att_r)


def _tc_edge_v(ea_pad, w1e_t):
    Epad = ea_pad.shape[0]
    BE = 4096

    def body(ea_ref, w_ref, v_ref):
        v_ref[...] = _dot(ea_ref[...], w_ref[...])

    return pl.pallas_call(
        body,
        grid=(Epad // BE,),
        in_specs=[
            pl.BlockSpec((BE, ea_pad.shape[1]), lambda i: (i, 0)),
            pl.BlockSpec(w1e_t.shape, lambda i: (0, 0)),
        ],
        out_specs=pl.BlockSpec((BE, H), lambda i: (i, 0)),
        out_shape=jax.ShapeDtypeStruct((Epad, H), F32),
    )(ea_pad, w1e_t)


def _tc_reduce(parts):
    _, NR, _ = parts.shape

    def body(p_ref, o_ref):
        o_ref[...] = jnp.sum(p_ref[...], axis=0)

    return pl.pallas_call(
        body,
        out_shape=jax.ShapeDtypeStruct((NR, 128), F32),
    )(parts)


def _tc_block(hacc, ssum, bias, xprev, wih, whh, bih, bhh,
              wnext_t, asrc, adst):
    N = xprev.shape[0]

    def body(hacc_ref, ss_ref, b_ref, xp_ref, wih_ref, whh_ref,
             bih_ref, bhh_ref, wn_ref, as_ref, ad_ref,
             xn_ref, xs_ref, s_ref, d_ref):
        h = _elu(hacc_ref[...] / (ss_ref[...] + 1e-16) + b_ref[...])
        xnew = jax.nn.relu(_gru(h, xp_ref[...], wih_ref[...], whh_ref[...],
                                bih_ref[...], bhh_ref[...]))
        xn_ref[...] = xnew
        xs = _dot(xnew, wn_ref[...])
        xs_ref[...] = xs
        s_ref[...] = _dot(xs, as_ref[...])
        d_ref[...] = _dot(xs, ad_ref[...])

    return pl.pallas_call(
        body,
        out_shape=[
            jax.ShapeDtypeStruct((N, H), F32),
            jax.ShapeDtypeStruct((N, H), F32),
            jax.ShapeDtypeStruct((N, 1), F32),
            jax.ShapeDtypeStruct((N, 1), F32),
        ],
    )(hacc, ssum, bias, xprev, wih, whh, bih, bhh, wnext_t, asrc, adst)


def _tc_mol(hacc, ssum, bias2, xprev, wih2, whh2, bih2, bhh2,
            batch_col, mol_wt, m_asrc, m_adst, mol_bias,
            mg_wih, mg_whh, mg_bih, mg_bhh, lin2_t, lin2_b, out_t, out_b):
    N = xprev.shape[0]
    G = 64
    OUT = out_t.shape[1]

    def body(hacc_ref, ss_ref, b2_ref, xp_ref, wih_ref, whh_ref,
             bih_ref, bhh_ref, bc_ref, mw_ref, mas_ref, mad_ref, mb_ref,
             gwih_ref, gwhh_ref, gbih_ref, gbhh_ref,
             l2_ref, l2b_ref, ow_ref, ob_ref, res_ref):
        h = _elu(hacc_ref[...] / (ss_ref[...] + 1e-16) + b2_ref[...])
        x4 = jax.nn.relu(_gru(h, xp_ref[...], wih_ref[...], whh_ref[...],
                              bih_ref[...], bhh_ref[...]))
        gid = lax.broadcasted_iota(jnp.int32, (N, G), 1)
        B = jnp.where(bc_ref[...] == gid, 1.0, 0.0).astype(F32)
        out = jax.nn.relu(_dotT(B, x4))
        xs = _dot(x4, mw_ref[...])
        s = _dot(xs, mas_ref[...])
        for _ in range(2):
            d = _dot(_dot(out, mw_ref[...]), mad_ref[...])
            dn = _dot(B, d)
            e = jnp.exp(_lk(s + dn))
            sg = _dotT(B, e)
            hm = _dotT(B, e * xs)
            hg = _elu(hm / (sg + 1e-16) + mb_ref[...])
            out = jax.nn.relu(_gru(hg, out, gwih_ref[...], gwhh_ref[...],
                                   gbih_ref[...], gbhh_ref[...]))
        r2 = _dot(out, l2_ref[...]) + l2b_ref[...]
        res_ref[...] = _dot(r2, ow_ref[...]) + ob_ref[...]

    return pl.pallas_call(
        body,
        out_shape=jax.ShapeDtypeStruct((G, OUT), F32),
    )(hacc, ssum, bias2, xprev, wih2, whh2, bih2, bhh2, batch_col,
      mol_wt, m_asrc, m_adst, mol_bias, mg_wih, mg_whh, mg_bih, mg_bhh,
      lin2_t, lin2_b, out_t, out_b)


def _mesh():
    return plsc.VectorSubcoreMesh(core_axis_name="c", subcore_axis_name="s")


def _sc_gat_scalar(sv2, dv2, src3, dst3, znr, E):
    NR = sv2.shape[0]
    NCH = src3.shape[1]
    EPT = NCH * 128

    @functools.partial(
        pl.kernel, mesh=_mesh(),
        compiler_params=pltpu.CompilerParams(needs_layout_passes=False),
        out_type=[
            jax.ShapeDtypeStruct((NW, NCH, 128), F32),
            jax.ShapeDtypeStruct((NW, NR, 128), F32),
        ],
        scratch_types=[
            pltpu.VMEM((NR, 128), F32),
            pltpu.VMEM((NR, 128), F32),
            pltpu.VMEM((NCH, 128), jnp.int32),
            pltpu.VMEM((NCH, 128), jnp.int32),
            pltpu.VMEM((NCH, 128), F32),
            pltpu.VMEM((NR, 128), F32),
            pltpu.SemaphoreType.DMA,
        ],
    )
    def k(sv_hbm, dv_hbm, src_hbm, dst_hbm, znr_hbm,
          e_out, ssum_out,
          sv_v, dv_v, src_v, dst_v, e_t, ssum_t, sem):
        c = lax.axis_index("c")
        s = lax.axis_index("s")
        wid = c * NS + s
        pltpu.sync_copy(sv_hbm, sv_v)
        pltpu.sync_copy(dv_hbm, dv_v)
        pltpu.sync_copy(src_hbm.at[wid], src_v)
        pltpu.sync_copy(dst_hbm.at[wid], dst_v)
        pltpu.sync_copy(znr_hbm, ssum_t)
        base = wid * EPT
        lane = lax.broadcasted_iota(jnp.int32, (16,), 0)

        def group(j, g, _):
            sl = pl.ds(g * 16, 16)
            src16 = src_v[j, sl]
            dst16 = dst_v[j, sl]
            s16 = plsc.load_gather(
                sv_v, [lax.shift_right_logical(src16, 7),
                       jnp.bitwise_and(src16, 127)])
            d16 = plsc.load_gather(
                dv_v, [lax.shift_right_logical(dst16, 7),
                       jnp.bitwise_and(dst16, 127)])
            e16 = jnp.exp(_lk(s16 + d16))
            eid = base + j * 128 + g * 16 + lane
            e16 = jnp.where(eid < E, e16, 0.0)
            e_t[j, sl] = e16
            plsc.addupdate_scatter(
                ssum_t, [lax.shift_right_logical(dst16, 7),
                         jnp.bitwise_and(dst16, 127)], e16)
            return 0

        def chunk(j, _):
            lax.fori_loop(0, 8, functools.partial(group, j), 0)
            return 0

        lax.fori_loop(0, NCH, chunk, 0)
        pltpu.sync_copy(e_t, e_out.at[wid])
        pltpu.sync_copy(ssum_t, ssum_out.at[wid])

    return k(sv2, dv2, src3, dst3, znr)


def _sc_gate_scalar(u, rv2, attl, vpad, src3, dst3, znr, E):
    NR = rv2.shape[0]
    NCH = src3.shape[1]
    EPT = NCH * 128

    @functools.partial(
        pl.kernel, mesh=_mesh(),
        compiler_params=pltpu.CompilerParams(needs_layout_passes=False),
        out_type=[
            jax.ShapeDtypeStruct((NW, NCH, 128), F32),
            jax.ShapeDtypeStruct((NW, NR, 128), F32),
        ],
        scratch_types=[
            pltpu.VMEM((NR, 128), F32),
            pltpu.VMEM((8, 16), F32),
            pltpu.VMEM((NCH, 128), jnp.int32),
            pltpu.VMEM((NCH, 128), jnp.int32),
            pltpu.VMEM((128, H), F32),
            pltpu.VMEM((128, H), F32),
            pltpu.VMEM((NCH, 128), F32),
            pltpu.VMEM((NR, 128), F32),
            pltpu.SemaphoreType.DMA,
        ],
    )
    def k(u_hbm, rv_hbm, al_hbm, v_hbm, src_hbm, dst_hbm, znr_hbm,
          e_out, ssum_out,
          rv_v, al_v, src_v, dst_v, ru_v, v_v, e_t, ssum_t, sem):
        c = lax.axis_index("c")
        s = lax.axis_index("s")
        wid = c * NS + s
        pltpu.sync_copy(rv_hbm, rv_v)
        pltpu.sync_copy(al_hbm, al_v)
        pltpu.sync_copy(src_hbm.at[wid], src_v)
        pltpu.sync_copy(dst_hbm.at[wid], dst_v)
        pltpu.sync_copy(znr_hbm, ssum_t)
        base = wid * EPT
        lane = lax.broadcasted_iota(jnp.int32, (16,), 0)
        alc = [al_v[kk, :] for kk in range(8)]

        def group(j, g, _):
            sl = pl.ds(g * 16, 16)
            dst16 = dst_v[j, sl]
            r16 = plsc.load_gather(
                rv_v, [lax.shift_right_logical(dst16, 7),
                       jnp.bitwise_and(dst16, 127)])
            ea16 = jnp.zeros((16,), F32)
            for i in range(16):
                ri = g * 16 + i
                acc = jnp.zeros((16,), F32)
                for kk in range(8):
                    ks = pl.ds(kk * 16, 16)
                    acc = acc + _lk(ru_v[ri, ks] + v_v[ri, ks]) * alc[kk]
                ea16 = jnp.where(lane == i, jnp.sum(acc), ea16)
            e16 = jnp.exp(_lk(ea16 + r16))
            eid = base + j * 128 + g * 16 + lane
            e16 = jnp.where(eid < E, e16, 0.0)
            e_t[j, sl] = e16
            plsc.addupdate_scatter(
                ssum_t, [lax.shift_right_logical(dst16, 7),
                         jnp.bitwise_and(dst16, 127)], e16)
            return 0

        def chunk(j, _):
            cp1 = pltpu.async_copy(u_hbm.at[src_v.at[j]], ru_v, sem)
            cp2 = pltpu.async_copy(v_hbm.at[pl.ds(base + j * 128, 128)],
                                   v_v, sem)
            cp1.wait()
            cp2.wait()
            lax.fori_loop(0, 8, functools.partial(group, j), 0)
            return 0

        lax.fori_loop(0, NCH, chunk, 0)
        pltpu.sync_copy(e_t, e_out.at[wid])
        pltpu.sync_copy(ssum_t, ssum_out.at[wid])

    return k(u, rv2, attl, vpad, src3, dst3, znr)


def _sc_rows(mtab, srcf, dstf, ef, zn4):
    NFR = mtab.shape[1]
    NR = NFR // FPT
    NCHA = srcf.shape[0]
    NSUP = NCHA // 16

    @functools.partial(
        pl.kernel, mesh=_mesh(),
        compiler_params=pltpu.CompilerParams(needs_layout_passes=False),
        out_type=jax.ShapeDtypeStruct((NW, NFR, 128), F32),
        scratch_types=[
            pltpu.VMEM((NFR, 128), F32),
            pltpu.VMEM((NFR, 128), F32),
            pltpu.VMEM((16, 128), jnp.int32),
            pltpu.VMEM((16, 128), jnp.int32),
            pltpu.VMEM((16, 128), F32),
            pltpu.SemaphoreType.DMA,
        ],
    )
    def k(mt_hbm, src_hbm, dst_hbm, e_hbm, zn4_hbm,
          acc_out,
          m4_v, acc_v, sbuf, dbuf, ebuf, sem):
        c = lax.axis_index("c")
        s = lax.axis_index("s")
        wid = c * NS + s
        pltpu.sync_copy(mt_hbm.at[wid], m4_v)
        pltpu.sync_copy(zn4_hbm, acc_v)

        def row(r):
            for g in range(8):
                sl = pl.ds(g * 16, 16)
                src16 = sbuf[r, sl]
                dst16 = dbuf[r, sl]
                e16 = ebuf[r, sl]
                sr = lax.shift_right_logical(src16, 7)
                scol = jnp.bitwise_and(src16, 127)
                dr = lax.shift_right_logical(dst16, 7)
                dcol = jnp.bitwise_and(dst16, 127)
                for fi in range(FPT):
                    v = plsc.load_gather(m4_v, [sr + (fi * NR), scol])
                    plsc.addupdate_scatter(acc_v,
                                           [dr + (fi * NR), dcol],
                                           v * e16)

        def sup(t, _):
            sl16 = pl.ds(t * 16, 16)
            pltpu.sync_copy(src_hbm.at[sl16], sbuf)
            pltpu.sync_copy(dst_hbm.at[sl16], dbuf)
            pltpu.sync_copy(e_hbm.at[sl16], ebuf)
            plsc.parallel_loop(0, 16, unroll=2)(row)
            return 0

        lax.fori_loop(0, NSUP, sup, 0)
        pltpu.sync_copy(acc_v, acc_out.at[wid])

    return k(mtab, srcf, dstf, ef, zn4)


def kernel(x, edge_index, edge_attr, batch, params):
    p = params
    N = x.shape[0]
    E = edge_index.shape[1]
    DE = edge_attr.shape[1]

    EPT = ((E + NW * 128 - 1) // (NW * 128)) * 128
    Epad = EPT * NW
    NCH = EPT // 128
    NCHA = Epad // 128
    src3 = jnp.concatenate([edge_index[0],
                            jnp.zeros((Epad - E,), jnp.int32)]).reshape(NW, NCH, 128)
    dst3 = jnp.concatenate([edge_index[1],
                            jnp.zeros((Epad - E,), jnp.int32)]).reshape(NW, NCH, 128)
    srcf = src3.reshape(NCHA, 128)
    dstf = dst3.reshape(NCHA, 128)
    ea_pad = jnp.concatenate(
        [edge_attr, jnp.zeros((Epad - E, DE), F32)], axis=0)

    NR = (N + 127) // 128
    NP = NR * 128
    znr = jnp.zeros((NR, 128), F32)
    zn4 = jnp.zeros((FPT * NR, 128), F32)
    padN = lambda v: jnp.concatenate(
        [v, jnp.zeros((NP - N,), F32)]).reshape(NR, 128)

    def ftab(t):
        tt = jnp.pad(t.T, ((0, 0), (0, NP - N)))
        return tt.reshape(NW, FPT * NR, 128)

    r2 = lambda v: v.reshape(1, -1)
    c2 = lambda v: v.reshape(-1, 1)

    def finish(acc, parts):
        hacc = acc.reshape(H, NP)[:, :N].T
        ssum = _tc_reduce(parts).reshape(NP)[:N].reshape(N, 1)
        return hacc, ssum

    x1, u, m, r = _tc_node_pre(
        x, p['lin1_W'].T, r2(p['lin1_b']),
        p['gate_lin1_W'][:, :H].T, p['gate_lin2_W'].T, c2(p['gate_att_r']))

    vpad = _tc_edge_v(ea_pad, p['gate_lin1_W'][:, H:].T)

    ef, parts = _sc_gate_scalar(u, padN(r.reshape(-1)),
                                p['gate_att_l'].reshape(8, 16), vpad,
                                src3, dst3, znr, E)
    acc = _sc_rows(ftab(m), srcf, dstf, ef.reshape(NCHA, 128), zn4)
    hacc, ssum = finish(acc, parts)

    xcur = x1
    for i, (wih, whh, bih, bhh, bias_in) in enumerate([
            (p['gru0_Wih'], p['gru0_Whh'], p['gru0_bih'], p['gru0_bhh'],
             r2(p['gate_bias'])),
            (p['gru1_Wih'], p['gru1_Whh'], p['gru1_bih'], p['gru1_bhh'],
             r2(p['conv1_bias'])),
    ]):
        li = i + 1
        xcur, xs, sv, dv = _tc_block(
            hacc, ssum, bias_in, xcur, wih.T, whh.T, r2(bih), r2(bhh),
            p['conv%d_lin_W' % li].T, c2(p['conv%d_att_src' % li]),
            c2(p['conv%d_att_dst' % li]))
        ef, parts = _sc_gat_scalar(padN(sv.reshape(-1)), padN(dv.reshape(-1)),
                                   src3, dst3, znr, E)
        acc = _sc_rows(ftab(xs), srcf, dstf, ef.reshape(NCHA, 128), zn4)
        hacc, ssum = finish(acc, parts)

    out = _tc_mol(
        hacc, ssum, r2(p['conv2_bias']), xcur,
        p['gru2_Wih'].T, p['gru2_Whh'].T, r2(p['gru2_bih']), r2(p['gru2_bhh']),
        batch.reshape(-1, 1), p['mol_lin_W'].T, c2(p['mol_att_src']),
        c2(p['mol_att_dst']), r2(p['mol_bias']),
        p['molgru_Wih'].T, p['molgru_Whh'].T, r2(p['molgru_bih']),
        r2(p['molgru_bhh']),
        p['lin2_W'].T, r2(p['lin2_b']), p['out_W'].T, r2(p['out_b']))
    return out


# speedup vs baseline: 1.7534x; 1.0221x over previous
import functools

import jax
import jax.numpy as jnp
from jax import lax
from jax.experimental import pallas as pl
from jax.experimental.pallas import tpu as pltpu
from jax.experimental.pallas import tpu_sc as plsc

F32 = jnp.float32
H = 128
NC = 2
NS = 16
NW = NC * NS
FPT = H // NW


def _lk(x):
    return jnp.maximum(x, 0.01 * x)


def _elu(x):
    return jnp.where(x > 0, x, jnp.exp(x) - 1.0)


def _sig(x):
    return 1.0 / (1.0 + jnp.exp(-x))


def _dot(a, b):
    return jnp.dot(a, b, preferred_element_type=F32)


def _dotT(a, b):
    return lax.dot_general(a, b, (((0,), (0,)), ((), ())),
                           preferred_element_type=F32)


def _gru(inp, hid, wih, whh, bih, bhh):
    gi = _dot(inp, wih) + bih
    gh = _dot(hid, whh) + bhh
    r = _sig(gi[:, :H] + gh[:, :H])
    z = _sig(gi[:, H:2 * H] + gh[:, H:2 * H])
    n = jnp.tanh(gi[:, 2 * H:] + r * gh[:, 2 * H:])
    return (1.0 - z) * n + z * hid


def _tc_node_pre(x, lin1_wt, lin1_b, w1x_t, lin2_t, att_r):
    N = x.shape[0]

    def body(x_ref, w1_ref, b1_ref, wx_ref, w2_ref, ar_ref,
             x1_ref, u_ref, m_ref, r_ref):
        x1 = _lk(_dot(x_ref[...], w1_ref[...]) + b1_ref[...])
        x1_ref[...] = x1
        u_ref[...] = _dot(x1, wx_ref[...])
        m_ref[...] = _dot(x1, w2_ref[...])
        r_ref[...] = _dot(x1, ar_ref[...])

    return pl.pallas_call(
        body,
        out_shape=[
            jax.ShapeDtypeStruct((N, H), F32),
            jax.ShapeDtypeStruct((N, H), F32),
            jax.ShapeDtypeStruct((N, H), F32),
            jax.ShapeDtypeStruct((N, 1), F32),
        ],
    )(x, lin1_wt, lin1_b, w1x_t, lin2_t, att_r)


def _tc_edge_v(ea_pad, w1e_t):
    Epad = ea_pad.shape[0]
    BE = 4096

    def body(ea_ref, w_ref, v_ref):
        v_ref[...] = _dot(ea_ref[...], w_ref[...])

    return pl.pallas_call(
        body,
        grid=(Epad // BE,),
        in_specs=[
            pl.BlockSpec((BE, ea_pad.shape[1]), lambda i: (i, 0)),
            pl.BlockSpec(w1e_t.shape, lambda i: (0, 0)),
        ],
        out_specs=pl.BlockSpec((BE, H), lambda i: (i, 0)),
        out_shape=jax.ShapeDtypeStruct((Epad, H), F32),
    )(ea_pad, w1e_t)


def _tc_reduce(parts):
    _, NR, _ = parts.shape

    def body(p_ref, o_ref):
        o_ref[...] = jnp.sum(p_ref[...], axis=0)

    return pl.pallas_call(
        body,
        out_shape=jax.ShapeDtypeStruct((NR, 128), F32),
    )(parts)


def _tc_block(hacc, ssum, bias, xprev, wih, whh, bih, bhh,
              wnext_t, asrc, adst):
    N = xprev.shape[0]

    def body(hacc_ref, ss_ref, b_ref, xp_ref, wih_ref, whh_ref,
             bih_ref, bhh_ref, wn_ref, as_ref, ad_ref,
             xn_ref, xs_ref, s_ref, d_ref):
        h = _elu(hacc_ref[...] / (ss_ref[...] + 1e-16) + b_ref[...])
        xnew = jax.nn.relu(_gru(h, xp_ref[...], wih_ref[...], whh_ref[...],
                                bih_ref[...], bhh_ref[...]))
        xn_ref[...] = xnew
        xs = _dot(xnew, wn_ref[...])
        xs_ref[...] = xs
        s_ref[...] = _dot(xs, as_ref[...])
        d_ref[...] = _dot(xs, ad_ref[...])

    return pl.pallas_call(
        body,
        out_shape=[
            jax.ShapeDtypeStruct((N, H), F32),
            jax.ShapeDtypeStruct((N, H), F32),
            jax.ShapeDtypeStruct((N, 1), F32),
            jax.ShapeDtypeStruct((N, 1), F32),
        ],
    )(hacc, ssum, bias, xprev, wih, whh, bih, bhh, wnext_t, asrc, adst)


def _tc_mol(hacc, ssum, bias2, xprev, wih2, whh2, bih2, bhh2,
            batch_col, mol_wt, m_asrc, m_adst, mol_bias,
            mg_wih, mg_whh, mg_bih, mg_bhh, lin2_t, lin2_b, out_t, out_b):
    N = xprev.shape[0]
    G = 64
    OUT = out_t.shape[1]

    def body(hacc_ref, ss_ref, b2_ref, xp_ref, wih_ref, whh_ref,
             bih_ref, bhh_ref, bc_ref, mw_ref, mas_ref, mad_ref, mb_ref,
             gwih_ref, gwhh_ref, gbih_ref, gbhh_ref,
             l2_ref, l2b_ref, ow_ref, ob_ref, res_ref):
        h = _elu(hacc_ref[...] / (ss_ref[...] + 1e-16) + b2_ref[...])
        x4 = jax.nn.relu(_gru(h, xp_ref[...], wih_ref[...], whh_ref[...],
                              bih_ref[...], bhh_ref[...]))
        gid = lax.broadcasted_iota(jnp.int32, (N, G), 1)
        B = jnp.where(bc_ref[...] == gid, 1.0, 0.0).astype(F32)
        out = jax.nn.relu(_dotT(B, x4))
        xs = _dot(x4, mw_ref[...])
        s = _dot(xs, mas_ref[...])
        for _ in range(2):
            d = _dot(_dot(out, mw_ref[...]), mad_ref[...])
            dn = _dot(B, d)
            e = jnp.exp(_lk(s + dn))
            sg = _dotT(B, e)
            hm = _dotT(B, e * xs)
            hg = _elu(hm / (sg + 1e-16) + mb_ref[...])
            out = jax.nn.relu(_gru(hg, out, gwih_ref[...], gwhh_ref[...],
                                   gbih_ref[...], gbhh_ref[...]))
        r2 = _dot(out, l2_ref[...]) + l2b_ref[...]
        res_ref[...] = _dot(r2, ow_ref[...]) + ob_ref[...]

    return pl.pallas_call(
        body,
        out_shape=jax.ShapeDtypeStruct((G, OUT), F32),
    )(hacc, ssum, bias2, xprev, wih2, whh2, bih2, bhh2, batch_col,
      mol_wt, m_asrc, m_adst, mol_bias, mg_wih, mg_whh, mg_bih, mg_bhh,
      lin2_t, lin2_b, out_t, out_b)


def _mesh():
    return plsc.VectorSubcoreMesh(core_axis_name="c", subcore_axis_name="s")


def _sc_gat_scalar(sv2, dv2, src3, dst3, znr, E):
    NR = sv2.shape[0]
    NCH = src3.shape[1]
    EPT = NCH * 128

    @functools.partial(
        pl.kernel, mesh=_mesh(),
        compiler_params=pltpu.CompilerParams(needs_layout_passes=False),
        out_type=[
            jax.ShapeDtypeStruct((NW, NCH, 128), F32),
            jax.ShapeDtypeStruct((NW, NR, 128), F32),
        ],
        scratch_types=[
            pltpu.VMEM((NR, 128), F32),
            pltpu.VMEM((NR, 128), F32),
            pltpu.VMEM((NCH, 128), jnp.int32),
            pltpu.VMEM((NCH, 128), jnp.int32),
            pltpu.VMEM((NCH, 128), F32),
            pltpu.VMEM((NR, 128), F32),
            pltpu.SemaphoreType.DMA,
        ],
    )
    def k(sv_hbm, dv_hbm, src_hbm, dst_hbm, znr_hbm,
          e_out, ssum_out,
          sv_v, dv_v, src_v, dst_v, e_t, ssum_t, sem):
        c = lax.axis_index("c")
        s = lax.axis_index("s")
        wid = c * NS + s
        pltpu.sync_copy(sv_hbm, sv_v)
        pltpu.sync_copy(dv_hbm, dv_v)
        pltpu.sync_copy(src_hbm.at[wid], src_v)
        pltpu.sync_copy(dst_hbm.at[wid], dst_v)
        pltpu.sync_copy(znr_hbm, ssum_t)
        base = wid * EPT
        lane = lax.broadcasted_iota(jnp.int32, (16,), 0)

        def chunk(j):
            for g in range(8):
                sl = pl.ds(g * 16, 16)
                src16 = src_v[j, sl]
                dst16 = dst_v[j, sl]
                s16 = plsc.load_gather(
                    sv_v, [lax.shift_right_logical(src16, 7),
                           jnp.bitwise_and(src16, 127)])
                d16 = plsc.load_gather(
                    dv_v, [lax.shift_right_logical(dst16, 7),
                           jnp.bitwise_and(dst16, 127)])
                e16 = jnp.exp(_lk(s16 + d16))
                eid = base + j * 128 + g * 16 + lane
                e16 = jnp.where(eid < E, e16, 0.0)
                e_t[j, sl] = e16
                plsc.addupdate_scatter(
                    ssum_t, [lax.shift_right_logical(dst16, 7),
                             jnp.bitwise_and(dst16, 127)], e16)

        plsc.parallel_loop(0, NCH, unroll=2)(chunk)
        pltpu.sync_copy(e_t, e_out.at[wid])
        pltpu.sync_copy(ssum_t, ssum_out.at[wid])

    return k(sv2, dv2, src3, dst3, znr)


def _sc_gate_scalar(u, rv2, attl, vpad, src3, dst3, znr, E):
    NR = rv2.shape[0]
    NCH = src3.shape[1]
    EPT = NCH * 128

    @functools.partial(
        pl.kernel, mesh=_mesh(),
        compiler_params=pltpu.CompilerParams(needs_layout_passes=False),
        out_type=[
            jax.ShapeDtypeStruct((NW, NCH, 128), F32),
            jax.ShapeDtypeStruct((NW, NR, 128), F32),
        ],
        scratch_types=[
            pltpu.VMEM((NR, 128), F32),
            pltpu.VMEM((8, 16), F32),
            pltpu.VMEM((NCH, 128), jnp.int32),
            pltpu.VMEM((NCH, 128), jnp.int32),
            pltpu.VMEM((128, H), F32),
            pltpu.VMEM((128, H), F32),
            pltpu.VMEM((NCH, 128), F32),
            pltpu.VMEM((NR, 128), F32),
            pltpu.SemaphoreType.DMA,
        ],
    )
    def k(u_hbm, rv_hbm, al_hbm, v_hbm, src_hbm, dst_hbm, znr_hbm,
          e_out, ssum_out,
          rv_v, al_v, src_v, dst_v, ru_v, v_v, e_t, ssum_t, sem):
        c = lax.axis_index("c")
        s = lax.axis_index("s")
        wid = c * NS + s
        pltpu.sync_copy(rv_hbm, rv_v)
        pltpu.sync_copy(al_hbm, al_v)
        pltpu.sync_copy(src_hbm.at[wid], src_v)
        pltpu.sync_copy(dst_hbm.at[wid], dst_v)
        pltpu.sync_copy(znr_hbm, ssum_t)
        base = wid * EPT
        lane = lax.broadcasted_iota(jnp.int32, (16,), 0)
        alc = [al_v[kk, :] for kk in range(8)]

        def group(j, g):
            sl = pl.ds(g * 16, 16)
            dst16 = dst_v[j, sl]
            r16 = plsc.load_gather(
                rv_v, [lax.shift_right_logical(dst16, 7),
                       jnp.bitwise_and(dst16, 127)])
            ea16 = jnp.zeros((16,), F32)
            for i in range(16):
                ri = g * 16 + i
                acc = jnp.zeros((16,), F32)
                for kk in range(8):
                    ks = pl.ds(kk * 16, 16)
                    acc = acc + _lk(ru_v[ri, ks] + v_v[ri, ks]) * alc[kk]
                ea16 = jnp.where(lane == i, jnp.sum(acc), ea16)
            e16 = jnp.exp(_lk(ea16 + r16))
            eid = base + j * 128 + g * 16 + lane
            e16 = jnp.where(eid < E, e16, 0.0)
            e_t[j, sl] = e16
            plsc.addupdate_scatter(
                ssum_t, [lax.shift_right_logical(dst16, 7),
                         jnp.bitwise_and(dst16, 127)], e16)

        def chunk(j, _):
            cp1 = pltpu.async_copy(u_hbm.at[src_v.at[j]], ru_v, sem)
            cp2 = pltpu.async_copy(v_hbm.at[pl.ds(base + j * 128, 128)],
                                   v_v, sem)
            cp1.wait()
            cp2.wait()
            plsc.parallel_loop(0, 8)(functools.partial(group, j))
            return 0

        lax.fori_loop(0, NCH, chunk, 0)
        pltpu.sync_copy(e_t, e_out.at[wid])
        pltpu.sync_copy(ssum_t, ssum_out.at[wid])

    return k(u, rv2, attl, vpad, src3, dst3, znr)


def _sc_rows(mtab, srcf, dstf, ef, zn4):
    NFR = mtab.shape[1]
    NR = NFR // FPT
    NCHA = srcf.shape[0]
    NSUP = NCHA // 16

    @functools.partial(
        pl.kernel, mesh=_mesh(),
        compiler_params=pltpu.CompilerParams(needs_layout_passes=False),
        out_type=jax.ShapeDtypeStruct((NW, NFR, 128), F32),
        scratch_types=[
            pltpu.VMEM((NFR, 128), F32),
            pltpu.VMEM((NFR, 128), F32),
            pltpu.VMEM((16, 128), jnp.int32),
            pltpu.VMEM((16, 128), jnp.int32),
            pltpu.VMEM((16, 128), F32),
            pltpu.SemaphoreType.DMA,
        ],
    )
    def k(mt_hbm, src_hbm, dst_hbm, e_hbm, zn4_hbm,
          acc_out,
          m4_v, acc_v, sbuf, dbuf, ebuf, sem):
        c = lax.axis_index("c")
        s = lax.axis_index("s")
        wid = c * NS + s
        pltpu.sync_copy(mt_hbm.at[wid], m4_v)
        pltpu.sync_copy(zn4_hbm, acc_v)

        def row(r):
            for g in range(8):
                sl = pl.ds(g * 16, 16)
                src16 = sbuf[r, sl]
                dst16 = dbuf[r, sl]
                e16 = ebuf[r, sl]
                sr = lax.shift_right_logical(src16, 7)
                scol = jnp.bitwise_and(src16, 127)
                dr = lax.shift_right_logical(dst16, 7)
                dcol = jnp.bitwise_and(dst16, 127)
                for fi in range(FPT):
                    v = plsc.load_gather(m4_v, [sr + (fi * NR), scol])
                    plsc.addupdate_scatter(acc_v,
                                           [dr + (fi * NR), dcol],
                                           v * e16)

        def sup(t, _):
            sl16 = pl.ds(t * 16, 16)
            pltpu.sync_copy(src_hbm.at[sl16], sbuf)
            pltpu.sync_copy(dst_hbm.at[sl16], dbuf)
            pltpu.sync_copy(e_hbm.at[sl16], ebuf)
            plsc.parallel_loop(0, 16, unroll=2)(row)
            return 0

        lax.fori_loop(0, NSUP, sup, 0)
        pltpu.sync_copy(acc_v, acc_out.at[wid])

    return k(mtab, srcf, dstf, ef, zn4)


def kernel(x, edge_index, edge_attr, batch, params):
    p = params
    N = x.shape[0]
    E = edge_index.shape[1]
    DE = edge_attr.shape[1]

    EPT = ((E + NW * 128 - 1) // (NW * 128)) * 128
    Epad = EPT * NW
    NCH = EPT // 128
    NCHA = Epad // 128
    src3 = jnp.concatenate([edge_index[0],
                            jnp.zeros((Epad - E,), jnp.int32)]).reshape(NW, NCH, 128)
    dst3 = jnp.concatenate([edge_index[1],
                            jnp.zeros((Epad - E,), jnp.int32)]).reshape(NW, NCH, 128)
    srcf = src3.reshape(NCHA, 128)
    dstf = dst3.reshape(NCHA, 128)
    ea_pad = jnp.concatenate(
        [edge_attr, jnp.zeros((Epad - E, DE), F32)], axis=0)

    NR = (N + 127) // 128
    NP = NR * 128
    znr = jnp.zeros((NR, 128), F32)
    zn4 = jnp.zeros((FPT * NR, 128), F32)
    padN = lambda v: jnp.concatenate(
        [v, jnp.zeros((NP - N,), F32)]).reshape(NR, 128)

    def ftab(t):
        tt = jnp.pad(t.T, ((0, 0), (0, NP - N)))
        return tt.reshape(NW, FPT * NR, 128)

    r2 = lambda v: v.reshape(1, -1)
    c2 = lambda v: v.reshape(-1, 1)

    def finish(acc, parts):
        hacc = acc.reshape(H, NP)[:, :N].T
        ssum = _tc_reduce(parts).reshape(NP)[:N].reshape(N, 1)
        return hacc, ssum

    x1, u, m, r = _tc_node_pre(
        x, p['lin1_W'].T, r2(p['lin1_b']),
        p['gate_lin1_W'][:, :H].T, p['gate_lin2_W'].T, c2(p['gate_att_r']))

    vpad = _tc_edge_v(ea_pad, p['gate_lin1_W'][:, H:].T)

    ef, parts = _sc_gate_scalar(u, padN(r.reshape(-1)),
                                p['gate_att_l'].reshape(8, 16), vpad,
                                src3, dst3, znr, E)
    acc = _sc_rows(ftab(m), srcf, dstf, ef.reshape(NCHA, 128), zn4)
    hacc, ssum = finish(acc, parts)

    xcur = x1
    for i, (wih, whh, bih, bhh, bias_in) in enumerate([
            (p['gru0_Wih'], p['gru0_Whh'], p['gru0_bih'], p['gru0_bhh'],
             r2(p['gate_bias'])),
            (p['gru1_Wih'], p['gru1_Whh'], p['gru1_bih'], p['gru1_bhh'],
             r2(p['conv1_bias'])),
    ]):
        li = i + 1
        xcur, xs, sv, dv = _tc_block(
            hacc, ssum, bias_in, xcur, wih.T, whh.T, r2(bih), r2(bhh),
            p['conv%d_lin_W' % li].T, c2(p['conv%d_att_src' % li]),
            c2(p['conv%d_att_dst' % li]))
        ef, parts = _sc_gat_scalar(padN(sv.reshape(-1)), padN(dv.reshape(-1)),
                                   src3, dst3, znr, E)
        acc = _sc_rows(ftab(xs), srcf, dstf, ef.reshape(NCHA, 128), zn4)
        hacc, ssum = finish(acc, parts)

    out = _tc_mol(
        hacc, ssum, r2(p['conv2_bias']), xcur,
        p['gru2_Wih'].T, p['gru2_Whh'].T, r2(p['gru2_bih']), r2(p['gru2_bhh']),
        batch.reshape(-1, 1), p['mol_lin_W'].T, c2(p['mol_att_src']),
        c2(p['mol_att_dst']), r2(p['mol_bias']),
        p['molgru_Wih'].T, p['molgru_Whh'].T, r2(p['molgru_bih']),
        r2(p['molgru_bhh']),
        p['lin2_W'].T, r2(p['lin2_b']), p['out_W'].T, r2(p['out_b']))
    return out


# speedup vs baseline: 2.2198x; 1.2660x over previous
import functools

import jax
import jax.numpy as jnp
from jax import lax
from jax.experimental import pallas as pl
from jax.experimental.pallas import tpu as pltpu
from jax.experimental.pallas import tpu_sc as plsc

F32 = jnp.float32
H = 128
NC = 2
NS = 16
NW = NC * NS
FPT = H // NW


def _lk(x):
    return jnp.maximum(x, 0.01 * x)


def _elu(x):
    return jnp.where(x > 0, x, jnp.exp(x) - 1.0)


def _sig(x):
    return 1.0 / (1.0 + jnp.exp(-x))


def _dot(a, b):
    return jnp.dot(a, b, preferred_element_type=F32)


def _dotT(a, b):
    return lax.dot_general(a, b, (((0,), (0,)), ((), ())),
                           preferred_element_type=F32)


def _gru(inp, hid, wih, whh, bih, bhh):
    gi = _dot(inp, wih) + bih
    gh = _dot(hid, whh) + bhh
    r = _sig(gi[:, :H] + gh[:, :H])
    z = _sig(gi[:, H:2 * H] + gh[:, H:2 * H])
    n = jnp.tanh(gi[:, 2 * H:] + r * gh[:, 2 * H:])
    return (1.0 - z) * n + z * hid


def _tc_node_pre(x, lin1_wt, lin1_b, w1x_t, lin2_t, att_r):
    N = x.shape[0]

    def body(x_ref, w1_ref, b1_ref, wx_ref, w2_ref, ar_ref,
             x1_ref, u_ref, m_ref, r_ref):
        x1 = _lk(_dot(x_ref[...], w1_ref[...]) + b1_ref[...])
        x1_ref[...] = x1
        u_ref[...] = _dot(x1, wx_ref[...])
        m_ref[...] = _dot(x1, w2_ref[...])
        r_ref[...] = _dot(x1, ar_ref[...])

    return pl.pallas_call(
        body,
        out_shape=[
            jax.ShapeDtypeStruct((N, H), F32),
            jax.ShapeDtypeStruct((N, H), F32),
            jax.ShapeDtypeStruct((N, H), F32),
            jax.ShapeDtypeStruct((N, 1), F32),
        ],
    )(x, lin1_wt, lin1_b, w1x_t, lin2_t, att_r)


def _tc_edge_v(ea_pad, w1e_t):
    Epad = ea_pad.shape[0]
    BE = 4096

    def body(ea_ref, w_ref, v_ref):
        v_ref[...] = _dot(ea_ref[...], w_ref[...])

    return pl.pallas_call(
        body,
        grid=(Epad // BE,),
        in_specs=[
            pl.BlockSpec((BE, ea_pad.shape[1]), lambda i: (i, 0)),
            pl.BlockSpec(w1e_t.shape, lambda i: (0, 0)),
        ],
        out_specs=pl.BlockSpec((BE, H), lambda i: (i, 0)),
        out_shape=jax.ShapeDtypeStruct((Epad, H), F32),
    )(ea_pad, w1e_t)


def _tc_reduce(parts):
    _, NR, _ = parts.shape

    def body(p_ref, o_ref):
        o_ref[...] = jnp.sum(p_ref[...], axis=0)

    return pl.pallas_call(
        body,
        out_shape=jax.ShapeDtypeStruct((NR, 128), F32),
    )(parts)


def _tc_block(hacc, ssum, bias, xprev, wih, whh, bih, bhh,
              wnext_t, asrc, adst):
    N = xprev.shape[0]

    def body(hacc_ref, ss_ref, b_ref, xp_ref, wih_ref, whh_ref,
             bih_ref, bhh_ref, wn_ref, as_ref, ad_ref,
             xn_ref, xs_ref, s_ref, d_ref):
        h = _elu(hacc_ref[...] / (ss_ref[...] + 1e-16) + b_ref[...])
        xnew = jax.nn.relu(_gru(h, xp_ref[...], wih_ref[...], whh_ref[...],
                                bih_ref[...], bhh_ref[...]))
        xn_ref[...] = xnew
        xs = _dot(xnew, wn_ref[...])
        xs_ref[...] = xs
        s_ref[...] = _dot(xs, as_ref[...])
        d_ref[...] = _dot(xs, ad_ref[...])

    return pl.pallas_call(
        body,
        out_shape=[
            jax.ShapeDtypeStruct((N, H), F32),
            jax.ShapeDtypeStruct((N, H), F32),
            jax.ShapeDtypeStruct((N, 1), F32),
            jax.ShapeDtypeStruct((N, 1), F32),
        ],
    )(hacc, ssum, bias, xprev, wih, whh, bih, bhh, wnext_t, asrc, adst)


def _tc_mol(hacc, ssum, bias2, xprev, wih2, whh2, bih2, bhh2,
            batch_col, mol_wt, m_asrc, m_adst, mol_bias,
            mg_wih, mg_whh, mg_bih, mg_bhh, lin2_t, lin2_b, out_t, out_b):
    N = xprev.shape[0]
    G = 64
    OUT = out_t.shape[1]

    def body(hacc_ref, ss_ref, b2_ref, xp_ref, wih_ref, whh_ref,
             bih_ref, bhh_ref, bc_ref, mw_ref, mas_ref, mad_ref, mb_ref,
             gwih_ref, gwhh_ref, gbih_ref, gbhh_ref,
             l2_ref, l2b_ref, ow_ref, ob_ref, res_ref):
        h = _elu(hacc_ref[...] / (ss_ref[...] + 1e-16) + b2_ref[...])
        x4 = jax.nn.relu(_gru(h, xp_ref[...], wih_ref[...], whh_ref[...],
                              bih_ref[...], bhh_ref[...]))
        gid = lax.broadcasted_iota(jnp.int32, (N, G), 1)
        B = jnp.where(bc_ref[...] == gid, 1.0, 0.0).astype(F32)
        out = jax.nn.relu(_dotT(B, x4))
        xs = _dot(x4, mw_ref[...])
        s = _dot(xs, mas_ref[...])
        for _ in range(2):
            d = _dot(_dot(out, mw_ref[...]), mad_ref[...])
            dn = _dot(B, d)
            e = jnp.exp(_lk(s + dn))
            sg = _dotT(B, e)
            hm = _dotT(B, e * xs)
            hg = _elu(hm / (sg + 1e-16) + mb_ref[...])
            out = jax.nn.relu(_gru(hg, out, gwih_ref[...], gwhh_ref[...],
                                   gbih_ref[...], gbhh_ref[...]))
        r2 = _dot(out, l2_ref[...]) + l2b_ref[...]
        res_ref[...] = _dot(r2, ow_ref[...]) + ob_ref[...]

    return pl.pallas_call(
        body,
        out_shape=jax.ShapeDtypeStruct((G, OUT), F32),
    )(hacc, ssum, bias2, xprev, wih2, whh2, bih2, bhh2, batch_col,
      mol_wt, m_asrc, m_adst, mol_bias, mg_wih, mg_whh, mg_bih, mg_bhh,
      lin2_t, lin2_b, out_t, out_b)


def _mesh():
    return plsc.VectorSubcoreMesh(core_axis_name="c", subcore_axis_name="s")


def _sc_gat_scalar(sv2, dv2, src3, dst3, znr, E):
    NR = sv2.shape[0]
    NCH = src3.shape[1]
    EPT = NCH * 128

    @functools.partial(
        pl.kernel, mesh=_mesh(),
        compiler_params=pltpu.CompilerParams(needs_layout_passes=False),
        out_type=[
            jax.ShapeDtypeStruct((NW, NCH, 128), F32),
            jax.ShapeDtypeStruct((NW, NR, 128), F32),
        ],
        scratch_types=[
            pltpu.VMEM((NR, 128), F32),
            pltpu.VMEM((NR, 128), F32),
            pltpu.VMEM((NCH, 128), jnp.int32),
            pltpu.VMEM((NCH, 128), jnp.int32),
            pltpu.VMEM((NCH, 128), F32),
            pltpu.VMEM((NR, 128), F32),
            pltpu.SemaphoreType.DMA,
        ],
    )
    def k(sv_hbm, dv_hbm, src_hbm, dst_hbm, znr_hbm,
          e_out, ssum_out,
          sv_v, dv_v, src_v, dst_v, e_t, ssum_t, sem):
        c = lax.axis_index("c")
        s = lax.axis_index("s")
        wid = c * NS + s
        pltpu.sync_copy(sv_hbm, sv_v)
        pltpu.sync_copy(dv_hbm, dv_v)
        pltpu.sync_copy(src_hbm.at[wid], src_v)
        pltpu.sync_copy(dst_hbm.at[wid], dst_v)
        pltpu.sync_copy(znr_hbm, ssum_t)
        base = wid * EPT
        lane = lax.broadcasted_iota(jnp.int32, (16,), 0)

        def chunk(j):
            for g in range(8):
                sl = pl.ds(g * 16, 16)
                src16 = src_v[j, sl]
                dst16 = dst_v[j, sl]
                s16 = plsc.load_gather(
                    sv_v, [lax.shift_right_logical(src16, 7),
                           jnp.bitwise_and(src16, 127)])
                d16 = plsc.load_gather(
                    dv_v, [lax.shift_right_logical(dst16, 7),
                           jnp.bitwise_and(dst16, 127)])
                e16 = jnp.exp(_lk(s16 + d16))
                eid = base + j * 128 + g * 16 + lane
                e16 = jnp.where(eid < E, e16, 0.0)
                e_t[j, sl] = e16
                plsc.addupdate_scatter(
                    ssum_t, [lax.shift_right_logical(dst16, 7),
                             jnp.bitwise_and(dst16, 127)], e16)

        plsc.parallel_loop(0, NCH, unroll=2)(chunk)
        pltpu.sync_copy(e_t, e_out.at[wid])
        pltpu.sync_copy(ssum_t, ssum_out.at[wid])

    return k(sv2, dv2, src3, dst3, znr)


def _sc_gate_scalar(u, rv2, attl, vpad, src3, dst3, znr, E):
    NR = rv2.shape[0]
    NCH = src3.shape[1]
    EPT = NCH * 128

    @functools.partial(
        pl.kernel, mesh=_mesh(),
        compiler_params=pltpu.CompilerParams(needs_layout_passes=False),
        out_type=[
            jax.ShapeDtypeStruct((NW, NCH, 128), F32),
            jax.ShapeDtypeStruct((NW, NR, 128), F32),
        ],
        scratch_types=[
            pltpu.VMEM((NR, 128), F32),
            pltpu.VMEM((8, 16), F32),
            pltpu.VMEM((NCH, 128), jnp.int32),
            pltpu.VMEM((NCH, 128), jnp.int32),
            pltpu.VMEM((128, H), F32),
            pltpu.VMEM((128, H), F32),
            pltpu.VMEM((NCH, 128), F32),
            pltpu.VMEM((NR, 128), F32),
            pltpu.SemaphoreType.DMA,
        ],
    )
    def k(u_hbm, rv_hbm, al_hbm, v_hbm, src_hbm, dst_hbm, znr_hbm,
          e_out, ssum_out,
          rv_v, al_v, src_v, dst_v, ru_v, v_v, e_t, ssum_t, sem):
        c = lax.axis_index("c")
        s = lax.axis_index("s")
        wid = c * NS + s
        pltpu.sync_copy(rv_hbm, rv_v)
        pltpu.sync_copy(al_hbm, al_v)
        pltpu.sync_copy(src_hbm.at[wid], src_v)
        pltpu.sync_copy(dst_hbm.at[wid], dst_v)
        pltpu.sync_copy(znr_hbm, ssum_t)
        base = wid * EPT
        lane = lax.broadcasted_iota(jnp.int32, (16,), 0)
        alc = [al_v[kk, :] for kk in range(8)]

        def group(j, g):
            sl = pl.ds(g * 16, 16)
            dst16 = dst_v[j, sl]
            r16 = plsc.load_gather(
                rv_v, [lax.shift_right_logical(dst16, 7),
                       jnp.bitwise_and(dst16, 127)])
            ea16 = jnp.zeros((16,), F32)
            for i in range(16):
                ri = g * 16 + i
                acc = jnp.zeros((16,), F32)
                for kk in range(8):
                    ks = pl.ds(kk * 16, 16)
                    acc = acc + _lk(ru_v[ri, ks] + v_v[ri, ks]) * alc[kk]
                ea16 = jnp.where(lane == i, jnp.sum(acc), ea16)
            e16 = jnp.exp(_lk(ea16 + r16))
            eid = base + j * 128 + g * 16 + lane
            e16 = jnp.where(eid < E, e16, 0.0)
            e_t[j, sl] = e16
            plsc.addupdate_scatter(
                ssum_t, [lax.shift_right_logical(dst16, 7),
                         jnp.bitwise_and(dst16, 127)], e16)

        def chunk(j, _):
            cp1 = pltpu.async_copy(u_hbm.at[src_v.at[j]], ru_v, sem)
            cp2 = pltpu.async_copy(v_hbm.at[pl.ds(base + j * 128, 128)],
                                   v_v, sem)
            cp1.wait()
            cp2.wait()
            plsc.parallel_loop(0, 8)(functools.partial(group, j))
            return 0

        lax.fori_loop(0, NCH, chunk, 0)
        pltpu.sync_copy(e_t, e_out.at[wid])
        pltpu.sync_copy(ssum_t, ssum_out.at[wid])

    return k(u, rv2, attl, vpad, src3, dst3, znr)


def _sc_rows(mtab, estream, zn4):
    NFR = mtab.shape[1]
    NR = NFR // FPT
    NCHA = estream.shape[0] // 3
    NSUP = NCHA // 16

    @functools.partial(
        pl.kernel, mesh=_mesh(),
        compiler_params=pltpu.CompilerParams(needs_layout_passes=False),
        out_type=jax.ShapeDtypeStruct((NW, NFR, 128), F32),
        scratch_types=[
            pltpu.VMEM((NFR, 128), F32),
            pltpu.VMEM((NFR, 128), F32),
            pltpu.VMEM((48, 128), jnp.int32),
            pltpu.SemaphoreType.DMA,
        ],
    )
    def k(mt_hbm, es_hbm, zn4_hbm,
          acc_out,
          m4_v, acc_v, ebuf, sem):
        c = lax.axis_index("c")
        s = lax.axis_index("s")
        wid = c * NS + s
        pltpu.sync_copy(mt_hbm.at[wid], m4_v)
        pltpu.sync_copy(zn4_hbm, acc_v)

        def row(r):
            for g in range(8):
                sl = pl.ds(g * 16, 16)
                src16 = ebuf[r * 3, sl]
                dst16 = ebuf[r * 3 + 1, sl]
                e16 = plsc.bitcast(ebuf[r * 3 + 2, sl], F32)
                sr = lax.shift_right_logical(src16, 7)
                scol = jnp.bitwise_and(src16, 127)
                dr = lax.shift_right_logical(dst16, 7)
                dcol = jnp.bitwise_and(dst16, 127)
                for fi in range(FPT):
                    v = plsc.load_gather(m4_v, [sr + (fi * NR), scol])
                    plsc.addupdate_scatter(acc_v,
                                           [dr + (fi * NR), dcol],
                                           v * e16)

        def sup(t, _):
            pltpu.sync_copy(es_hbm.at[pl.ds(t * 48, 48)], ebuf)
            plsc.parallel_loop(0, 16, unroll=4)(row)
            return 0

        lax.fori_loop(0, NSUP, sup, 0)
        pltpu.sync_copy(acc_v, acc_out.at[wid])

    return k(mtab, estream, zn4)


def kernel(x, edge_index, edge_attr, batch, params):
    p = params
    N = x.shape[0]
    E = edge_index.shape[1]
    DE = edge_attr.shape[1]

    EPT = ((E + NW * 128 - 1) // (NW * 128)) * 128
    Epad = EPT * NW
    NCH = EPT // 128
    NCHA = Epad // 128
    src3 = jnp.concatenate([edge_index[0],
                            jnp.zeros((Epad - E,), jnp.int32)]).reshape(NW, NCH, 128)
    dst3 = jnp.concatenate([edge_index[1],
                            jnp.zeros((Epad - E,), jnp.int32)]).reshape(NW, NCH, 128)
    srcf = src3.reshape(NCHA, 128)
    dstf = dst3.reshape(NCHA, 128)
    base_sd = jnp.concatenate([srcf[:, None, :], dstf[:, None, :]], axis=1)
    mk_es = lambda ef: jnp.concatenate(
        [base_sd,
         jax.lax.bitcast_convert_type(ef.reshape(NCHA, 128),
                                      jnp.int32)[:, None, :]],
        axis=1).reshape(NCHA * 3, 128)
    ea_pad = jnp.concatenate(
        [edge_attr, jnp.zeros((Epad - E, DE), F32)], axis=0)

    NR = (N + 127) // 128
    NP = NR * 128
    znr = jnp.zeros((NR, 128), F32)
    zn4 = jnp.zeros((FPT * NR, 128), F32)
    padN = lambda v: jnp.concatenate(
        [v, jnp.zeros((NP - N,), F32)]).reshape(NR, 128)

    def ftab(t):
        tt = jnp.pad(t.T, ((0, 0), (0, NP - N)))
        return tt.reshape(NW, FPT * NR, 128)

    r2 = lambda v: v.reshape(1, -1)
    c2 = lambda v: v.reshape(-1, 1)

    def finish(acc, parts):
        hacc = acc.reshape(H, NP)[:, :N].T
        ssum = _tc_reduce(parts).reshape(NP)[:N].reshape(N, 1)
        return hacc, ssum

    x1, u, m, r = _tc_node_pre(
        x, p['lin1_W'].T, r2(p['lin1_b']),
        p['gate_lin1_W'][:, :H].T, p['gate_lin2_W'].T, c2(p['gate_att_r']))

    vpad = _tc_edge_v(ea_pad, p['gate_lin1_W'][:, H:].T)

    ef, parts = _sc_gate_scalar(u, padN(r.reshape(-1)),
                                p['gate_att_l'].reshape(8, 16), vpad,
                                src3, dst3, znr, E)
    acc = _sc_rows(ftab(m), mk_es(ef), zn4)
    hacc, ssum = finish(acc, parts)

    xcur = x1
    for i, (wih, whh, bih, bhh, bias_in) in enumerate([
            (p['gru0_Wih'], p['gru0_Whh'], p['gru0_bih'], p['gru0_bhh'],
             r2(p['gate_bias'])),
            (p['gru1_Wih'], p['gru1_Whh'], p['gru1_bih'], p['gru1_bhh'],
             r2(p['conv1_bias'])),
    ]):
        li = i + 1
        xcur, xs, sv, dv = _tc_block(
            hacc, ssum, bias_in, xcur, wih.T, whh.T, r2(bih), r2(bhh),
            p['conv%d_lin_W' % li].T, c2(p['conv%d_att_src' % li]),
            c2(p['conv%d_att_dst' % li]))
        ef, parts = _sc_gat_scalar(padN(sv.reshape(-1)), padN(dv.reshape(-1)),
                                   src3, dst3, znr, E)
        acc = _sc_rows(ftab(xs), mk_es(ef), zn4)
        hacc, ssum = finish(acc, parts)

    out = _tc_mol(
        hacc, ssum, r2(p['conv2_bias']), xcur,
        p['gru2_Wih'].T, p['gru2_Whh'].T, r2(p['gru2_bih']), r2(p['gru2_bhh']),
        batch.reshape(-1, 1), p['mol_lin_W'].T, c2(p['mol_att_src']),
        c2(p['mol_att_dst']), r2(p['mol_bias']),
        p['molgru_Wih'].T, p['molgru_Whh'].T, r2(p['molgru_bih']),
        r2(p['molgru_bhh']),
        p['lin2_W'].T, r2(p['lin2_b']), p['out_W'].T, r2(p['out_b']))
    return out
